# baseline (speedup 1.0000x reference)
import functools

import jax
import jax.numpy as jnp
from jax import lax
from jax.experimental import pallas as pl
from jax.experimental.pallas import tpu as pltpu
from jax.experimental.pallas import tpu_sc as plsc

NUM_STATES = 100000
NUM_RTG = 8
NUM_ACTIONS = 64
MIN_RTG = 1.0
MAX_RTG = 256.0
B, T = 4096, 50
N = B * T
NC, NS, L = 2, 16, 16
NW = NC * NS
N_PER_W = N // NW
CHUNK = 128
N_CHUNKS = N_PER_W // CHUNK


def _prep_body(states_ref, rewards_ref, out_ref):
    r = rewards_ref[...]
    r = jnp.where(r < 0.0, 0.0, r)
    ri = r.astype(jnp.int32)
    x = ri
    d = 1
    while d < T:
        shifted = jnp.concatenate(
            [x[:, d:], jnp.zeros((B, d), jnp.int32)], axis=1)
        x = x + shifted
        d *= 2
    rtg = ((x.astype(jnp.float32) - MIN_RTG) / (MAX_RTG - MIN_RTG)
           * NUM_RTG).astype(jnp.int32)
    st = states_ref[...]
    st = jnp.where(st < 0, 0, st)
    out_ref[...] = st * NUM_RTG + rtg


_prep = pl.pallas_call(
    _prep_body,
    out_shape=jax.ShapeDtypeStruct((B, T), jnp.int32),
)


def _log16(x):
    bits = plsc.bitcast(x, jnp.int32)
    e = ((bits >> 23) & 0xFF) - 127
    mbits = (bits & 0x7FFFFF) | (127 << 23)
    m = plsc.bitcast(mbits, jnp.float32)
    big = m > 1.4142135
    m = jnp.where(big, m * 0.5, m)
    e = jnp.where(big, e + 1, e)
    t = (m - 1.0) / (m + 1.0)
    t2 = t * t
    p = t * (2.0 + t2 * (2.0 / 3.0 + t2 * (0.4 + t2 * (2.0 / 7.0
             + t2 * (2.0 / 9.0)))))
    return p + e.astype(jnp.float32) * 0.6931471805599453


_mesh = plsc.VectorSubcoreMesh(core_axis_name="c", subcore_axis_name="s")


@functools.partial(
    pl.kernel,
    mesh=_mesh,
    compiler_params=pltpu.CompilerParams(needs_layout_passes=False,
                                         use_tc_tiling_on_sc=False),
    out_type=jax.ShapeDtypeStruct((NW, L), jnp.float32),
    scratch_types=[
        pltpu.VMEM((N_CHUNKS, CHUNK), jnp.int32),
        pltpu.VMEM((N_PER_W,), jnp.int32),
        pltpu.VMEM((CHUNK, NUM_ACTIONS), jnp.float32),
        pltpu.VMEM((CHUNK, NUM_ACTIONS), jnp.float32),
        pltpu.VMEM((L,), jnp.float32),
        pltpu.SemaphoreType.DMA,
        pltpu.SemaphoreType.DMA,
    ],
)
def _sc_main(idx_hbm, act_hbm, pol_hbm, out_hbm,
             idx_v, act_v, buf0, buf1, acc_v, sem0, sem1):
    w = lax.axis_index("s") * NC + lax.axis_index("c")
    pltpu.sync_copy(idx_hbm.at[w], idx_v)
    pltpu.sync_copy(act_hbm.at[w], act_v)
    pltpu.async_copy(pol_hbm.at[idx_v.at[0]], buf0, sem0)
    pltpu.async_copy(pol_hbm.at[idx_v.at[1]], buf1, sem1)

    lanes = lax.broadcasted_iota(jnp.int32, (L,), 0)

    def compute(buf, base_e, acc):
        for g in range(CHUNK // L):
            lan = lanes + (g * L)
            a = act_v[pl.ds(base_e + g * L, L)]
            a = jnp.maximum(a, 0)

            def jbody(j4, s):
                col0 = j4 * 4
                for u in range(4):
                    cols = jnp.zeros((L,), jnp.int32) + (col0 + u)
                    v = plsc.load_gather(buf, [lan, cols])
                    s = s + jnp.exp(v)
                return s

            s = lax.fori_loop(0, NUM_ACTIONS // 4, jbody,
                              jnp.zeros((L,), jnp.float32))
            ll = plsc.load_gather(buf, [lan, a])
            acc = acc + (_log16(s) - ll)
        return acc

    def pair(cc, acc):
        c0 = cc * 2
        pltpu.make_async_copy(pol_hbm.at[idx_v.at[c0]], buf0, sem0).wait()
        acc = compute(buf0, c0 * CHUNK, acc)

        @pl.when(c0 + 2 < N_CHUNKS)
        def _():
            pltpu.async_copy(pol_hbm.at[idx_v.at[c0 + 2]], buf0, sem0)

        pltpu.make_async_copy(pol_hbm.at[idx_v.at[c0 + 1]], buf1, sem1).wait()
        acc = compute(buf1, (c0 + 1) * CHUNK, acc)

        @pl.when(c0 + 3 < N_CHUNKS)
        def _():
            pltpu.async_copy(pol_hbm.at[idx_v.at[c0 + 3]], buf1, sem1)

        return acc

    acc = lax.fori_loop(0, N_CHUNKS // 2, pair, jnp.zeros((L,), jnp.float32))
    acc_v[...] = acc
    pltpu.sync_copy(acc_v, out_hbm.at[w])


def _finish_body(p_ref, o_ref):
    o_ref[...] = (jnp.sum(p_ref[...]) * (1.0 / N)).reshape(1, 1)


_finish = pl.pallas_call(
    _finish_body,
    out_shape=jax.ShapeDtypeStruct((1, 1), jnp.float32),
)


def kernel(states, actions, rewards, policy):
    flat = _prep(states, rewards)
    idx = flat.reshape(NW, N_CHUNKS, CHUNK)
    act = actions.reshape(NW, N_PER_W)
    pol = policy.reshape(NUM_STATES * NUM_RTG, NUM_ACTIONS)
    partials = _sc_main(idx, act, pol)
    return _finish(partials)[0, 0]
